# baseline (speedup 1.0000x reference)
import jax
import jax.numpy as jnp
from jax.experimental import pallas as pl
from jax.experimental.pallas import tpu as pltpu

D = 768
NUM_HEADS = 12
HEAD_DIM = D // NUM_HEADS
ATT_SCALE = HEAD_DIM ** (-0.5)
FF = 3072
FF_TILE = 768
N_LAYERS = 12
LN_EPS = 1e-5
BB = 16
NCHUNK = 2
S = 80


def _norm(x):
    mu = jnp.mean(x, axis=-1, keepdims=True)
    var = jnp.mean(jnp.square(x - mu), axis=-1, keepdims=True)
    return (x - mu) * jax.lax.rsqrt(var + LN_EPS)


def _gelu_tanh(x):
    c = 0.7978845608028654
    c2 = c * 0.044715
    v = x * (c + c2 * (x * x))
    h = 0.5 * x
    return h + h * jnp.tanh(v)


def _attn_pass(xs, mask2d, w_qkv, b_qkv, bv1, wo, bo):
    nc = len(xs)
    qkvs = []
    for x in xs:
        rows = x.shape[0]
        cb = rows // S
        xn = (_norm(x) + bv1).astype(jnp.bfloat16)
        qkv = jnp.dot(xn, w_qkv, preferred_element_type=jnp.float32) + b_qkv
        qkvs.append(qkv.reshape(cb, S, 3 * D))

    ctx_heads = [[] for _ in range(nc)]
    for h in range(NUM_HEADS):
        lo = h * HEAD_DIM
        for c in range(nc):
            qkv = qkvs[c]
            qh = qkv[:, :, lo:lo + HEAD_DIM].astype(jnp.bfloat16)
            kh = qkv[:, :, D + lo:D + lo + HEAD_DIM].astype(jnp.bfloat16)
            vh = qkv[:, :, 2 * D + lo:2 * D + lo + HEAD_DIM].astype(jnp.bfloat16)
            sc = jax.lax.dot_general(qh, kh, (((2,), (2,)), ((0,), (0,))),
                                     preferred_element_type=jnp.float32)
            e = jnp.exp(sc + mask2d[None])
            p = (e / jnp.sum(e, axis=-1, keepdims=True)).astype(jnp.bfloat16)
            ctx_heads[c].append(jax.lax.dot_general(
                p, vh, (((2,), (1,)), ((0,), (0,))),
                preferred_element_type=jnp.float32).astype(jnp.bfloat16))

    outs = []
    for c in range(nc):
        rows = xs[c].shape[0]
        ctx = jnp.concatenate(ctx_heads[c], axis=-1)
        ctx = ctx.reshape(rows, D)
        attn = jnp.dot(ctx, wo, preferred_element_type=jnp.float32) + bo
        outs.append(xs[c] + attn)
    return outs


def _mlp_pass(xs, bv2, w1, b1, w2, b2):
    xn2s = [(_norm(x) + bv2).astype(jnp.bfloat16) for x in xs]
    accs = [x + b2 for x in xs]
    for t in range(FF // FF_TILE):
        fo = t * FF_TILE
        for c in range(len(xs)):
            ht = jnp.dot(xn2s[c], w1[:, fo:fo + FF_TILE],
                         preferred_element_type=jnp.float32) + b1[:, fo:fo + FF_TILE]
            ht = _gelu_tanh(ht).astype(jnp.bfloat16)
            accs[c] = accs[c] + jnp.dot(ht, w2[fo:fo + FF_TILE, :],
                                        preferred_element_type=jnp.float32)
    return accs


def _encoder_kernel(x_hbm, mask_ref, w_qkv_ref, b_qkv_ref, bv1_ref, wo_ref,
                    bo_ref, bv2_ref, w1_ref, b1_ref, w2_ref, b2_ref,
                    out_ref, dma_sem):
    layer = pl.program_id(1)

    @pl.when(layer == 0)
    def _():
        nb = pl.program_id(0)
        cp = pltpu.make_async_copy(x_hbm.at[pl.ds(nb * BB, BB)], out_ref,
                                   dma_sem)
        cp.start()
        cp.wait()

    mask2d = mask_ref[0, 0]
    cb = BB // NCHUNK
    xs = [out_ref[c * cb:(c + 1) * cb].reshape(cb * S, D)
          for c in range(NCHUNK)]
    xs = _attn_pass(xs, mask2d, w_qkv_ref[0], b_qkv_ref[0], bv1_ref[0],
                    wo_ref[0], bo_ref[0])
    ys = _mlp_pass(xs, bv2_ref[0], w1_ref[0], b1_ref[0], w2_ref[0],
                   b2_ref[0])
    for c in range(NCHUNK):
        out_ref[c * cb:(c + 1) * cb] = ys[c].reshape(cb, S, D)


def kernel(hidden, mask, ln1_g, ln1_b, qkv_w, qkv_b, wo, bo,
           ln2_g, ln2_b, w1, b1, w2, b2):
    B, S_, _ = hidden.shape
    nb = B // BB
    L = N_LAYERS

    cscale = jnp.concatenate(
        [jnp.full((D,), ATT_SCALE, jnp.float32),
         jnp.ones((2 * D,), jnp.float32)])
    w_qkv = (qkv_w * ln1_g[:, 0, :, None] * cscale).astype(jnp.bfloat16)
    b_qkv = qkv_b * cscale
    bv1 = ln1_b / ln1_g
    w1_f = (w1 * ln2_g[:, 0, :, None]).astype(jnp.bfloat16)
    bv2 = ln2_b / ln2_g
    wo_b = wo.astype(jnp.bfloat16)
    w2_b = w2.astype(jnp.bfloat16)

    return pl.pallas_call(
        _encoder_kernel,
        out_shape=jax.ShapeDtypeStruct((B, S_, D), jnp.float32),
        grid_spec=pltpu.PrefetchScalarGridSpec(
            num_scalar_prefetch=0,
            grid=(nb, N_LAYERS),
            in_specs=[
                pl.BlockSpec(memory_space=pl.ANY),
                pl.BlockSpec((1, 1, S_, S_), lambda b, l: (0, 0, 0, 0)),
                pl.BlockSpec((1, D, 3 * D), lambda b, l: (l, 0, 0)),
                pl.BlockSpec((1, 1, 3 * D), lambda b, l: (l, 0, 0)),
                pl.BlockSpec((1, 1, D), lambda b, l: (l, 0, 0)),
                pl.BlockSpec((1, D, D), lambda b, l: (l, 0, 0)),
                pl.BlockSpec((1, 1, D), lambda b, l: (l, 0, 0)),
                pl.BlockSpec((1, 1, D), lambda b, l: (l, 0, 0)),
                pl.BlockSpec((1, D, FF), lambda b, l: (l, 0, 0)),
                pl.BlockSpec((1, 1, FF), lambda b, l: (l, 0, 0)),
                pl.BlockSpec((1, FF, D), lambda b, l: (l, 0, 0)),
                pl.BlockSpec((1, 1, D), lambda b, l: (l, 0, 0)),
            ],
            out_specs=pl.BlockSpec((BB, S_, D), lambda b, l: (b, 0, 0)),
            scratch_shapes=[pltpu.SemaphoreType.DMA],
        ),
        compiler_params=pltpu.CompilerParams(
            dimension_semantics=("parallel", "arbitrary"),
            vmem_limit_bytes=56 * 1024 * 1024,
        ),
    )(hidden, mask, w_qkv, b_qkv, bv1, wo_b, bo, bv2, w1_f, b1, w2_b, b2)


# speedup vs baseline: 1.0041x; 1.0041x over previous
import jax
import jax.numpy as jnp
from jax.experimental import pallas as pl
from jax.experimental.pallas import tpu as pltpu

D = 768
NUM_HEADS = 12
HEAD_DIM = D // NUM_HEADS
ATT_SCALE = HEAD_DIM ** (-0.5)
FF = 3072
FF_TILE = 1536
N_LAYERS = 12
LN_EPS = 1e-5
BB = 16
NCHUNK = 2
S = 80


def _norm(x):
    mu = jnp.mean(x, axis=-1, keepdims=True)
    var = jnp.mean(jnp.square(x - mu), axis=-1, keepdims=True)
    return (x - mu) * jax.lax.rsqrt(var + LN_EPS)


def _gelu_tanh(x):
    c = 0.7978845608028654
    c2 = c * 0.044715
    v = x * (c + c2 * (x * x))
    h = 0.5 * x
    return h + h * jnp.tanh(v)


def _attn_pass(xs, mask2d, w_qkv, b_qkv, bv1, wo, bo):
    nc = len(xs)
    qkvs = []
    for x in xs:
        rows = x.shape[0]
        cb = rows // S
        xn = (_norm(x) + bv1).astype(jnp.bfloat16)
        qkv = jnp.dot(xn, w_qkv, preferred_element_type=jnp.float32) + b_qkv
        qkvs.append(qkv.reshape(cb, S, 3 * D))

    ctx_heads = [[] for _ in range(nc)]
    for h in range(NUM_HEADS):
        lo = h * HEAD_DIM
        for c in range(nc):
            qkv = qkvs[c]
            qh = qkv[:, :, lo:lo + HEAD_DIM].astype(jnp.bfloat16)
            kh = qkv[:, :, D + lo:D + lo + HEAD_DIM].astype(jnp.bfloat16)
            vh = qkv[:, :, 2 * D + lo:2 * D + lo + HEAD_DIM].astype(jnp.bfloat16)
            sc = jax.lax.dot_general(qh, kh, (((2,), (2,)), ((0,), (0,))),
                                     preferred_element_type=jnp.float32)
            e = jnp.exp(sc + mask2d[None])
            p = (e / jnp.sum(e, axis=-1, keepdims=True)).astype(jnp.bfloat16)
            ctx_heads[c].append(jax.lax.dot_general(
                p, vh, (((2,), (1,)), ((0,), (0,))),
                preferred_element_type=jnp.float32).astype(jnp.bfloat16))

    outs = []
    for c in range(nc):
        rows = xs[c].shape[0]
        ctx = jnp.concatenate(ctx_heads[c], axis=-1)
        ctx = ctx.reshape(rows, D)
        attn = jnp.dot(ctx, wo, preferred_element_type=jnp.float32) + bo
        outs.append(xs[c] + attn)
    return outs


def _mlp_pass(xs, bv2, w1, b1, w2, b2):
    xn2s = [(_norm(x) + bv2).astype(jnp.bfloat16) for x in xs]
    accs = [x + b2 for x in xs]
    for t in range(FF // FF_TILE):
        fo = t * FF_TILE
        for c in range(len(xs)):
            ht = jnp.dot(xn2s[c], w1[:, fo:fo + FF_TILE],
                         preferred_element_type=jnp.float32) + b1[:, fo:fo + FF_TILE]
            ht = _gelu_tanh(ht).astype(jnp.bfloat16)
            accs[c] = accs[c] + jnp.dot(ht, w2[fo:fo + FF_TILE, :],
                                        preferred_element_type=jnp.float32)
    return accs


def _encoder_kernel(x_hbm, mask_ref, w_qkv_ref, b_qkv_ref, bv1_ref, wo_ref,
                    bo_ref, bv2_ref, w1_ref, b1_ref, w2_ref, b2_ref,
                    out_ref, dma_sem):
    layer = pl.program_id(1)

    @pl.when(layer == 0)
    def _():
        nb = pl.program_id(0)
        cp = pltpu.make_async_copy(x_hbm.at[pl.ds(nb * BB, BB)], out_ref,
                                   dma_sem)
        cp.start()
        cp.wait()

    mask2d = mask_ref[0, 0]
    cb = BB // NCHUNK
    xs = [out_ref[c * cb:(c + 1) * cb].reshape(cb * S, D)
          for c in range(NCHUNK)]
    xs = _attn_pass(xs, mask2d, w_qkv_ref[0], b_qkv_ref[0], bv1_ref[0],
                    wo_ref[0], bo_ref[0])
    ys = _mlp_pass(xs, bv2_ref[0], w1_ref[0], b1_ref[0], w2_ref[0],
                   b2_ref[0])
    for c in range(NCHUNK):
        out_ref[c * cb:(c + 1) * cb] = ys[c].reshape(cb, S, D)


def kernel(hidden, mask, ln1_g, ln1_b, qkv_w, qkv_b, wo, bo,
           ln2_g, ln2_b, w1, b1, w2, b2):
    B, S_, _ = hidden.shape
    nb = B // BB
    L = N_LAYERS

    cscale = jnp.concatenate(
        [jnp.full((D,), ATT_SCALE, jnp.float32),
         jnp.ones((2 * D,), jnp.float32)])
    w_qkv = (qkv_w * ln1_g[:, 0, :, None] * cscale).astype(jnp.bfloat16)
    b_qkv = qkv_b * cscale
    bv1 = ln1_b / ln1_g
    w1_f = (w1 * ln2_g[:, 0, :, None]).astype(jnp.bfloat16)
    bv2 = ln2_b / ln2_g
    wo_b = wo.astype(jnp.bfloat16)
    w2_b = w2.astype(jnp.bfloat16)

    return pl.pallas_call(
        _encoder_kernel,
        out_shape=jax.ShapeDtypeStruct((B, S_, D), jnp.float32),
        grid_spec=pltpu.PrefetchScalarGridSpec(
            num_scalar_prefetch=0,
            grid=(nb, N_LAYERS),
            in_specs=[
                pl.BlockSpec(memory_space=pl.ANY),
                pl.BlockSpec((1, 1, S_, S_), lambda b, l: (0, 0, 0, 0)),
                pl.BlockSpec((1, D, 3 * D), lambda b, l: (l, 0, 0)),
                pl.BlockSpec((1, 1, 3 * D), lambda b, l: (l, 0, 0)),
                pl.BlockSpec((1, 1, D), lambda b, l: (l, 0, 0)),
                pl.BlockSpec((1, D, D), lambda b, l: (l, 0, 0)),
                pl.BlockSpec((1, 1, D), lambda b, l: (l, 0, 0)),
                pl.BlockSpec((1, 1, D), lambda b, l: (l, 0, 0)),
                pl.BlockSpec((1, D, FF), lambda b, l: (l, 0, 0)),
                pl.BlockSpec((1, 1, FF), lambda b, l: (l, 0, 0)),
                pl.BlockSpec((1, FF, D), lambda b, l: (l, 0, 0)),
                pl.BlockSpec((1, 1, D), lambda b, l: (l, 0, 0)),
            ],
            out_specs=pl.BlockSpec((BB, S_, D), lambda b, l: (b, 0, 0)),
            scratch_shapes=[pltpu.SemaphoreType.DMA],
        ),
        compiler_params=pltpu.CompilerParams(
            dimension_semantics=("parallel", "arbitrary"),
            vmem_limit_bytes=56 * 1024 * 1024,
        ),
    )(hidden, mask, w_qkv, b_qkv, bv1, wo_b, bo, bv2, w1_f, b1, w2_b, b2)
